# baseline (speedup 1.0000x reference)
import functools

import jax
import jax.numpy as jnp
from jax import lax
from jax.experimental import pallas as pl
from jax.experimental.pallas import tpu as pltpu
import jax.experimental.pallas.tpu_sc as plsc

N = 10000
E = 320000
D = 128
DP3 = 16
N_PAD = 10240
NC = 2
NS = 16
NW = NC * NS
EPW = 10240
E_PAD = NW * EPW
K = 80
NB = EPW // K
RPS = N_PAD // NS

_mesh = plsc.VectorSubcoreMesh(core_axis_name="c", subcore_axis_name="s")


CBUF = 4
CRING = NB // CBUF - 1


@functools.partial(
    pl.kernel,
    out_type=jax.ShapeDtypeStruct((NC, N_PAD), jnp.float32),
    mesh=_mesh,
    scratch_types=[
        pltpu.VMEM((CBUF, K), jnp.int32),
        pltpu.VMEM((K,), jnp.float32),
        pltpu.SemaphoreType.DMA((CBUF,)),
        pltpu.VMEM_SHARED((N_PAD,), jnp.float32),
    ],
)
def _sc_count(dst_hbm, zeros_hbm, out_hbm, dst_v, ones_v, isem, acc_sh):
    cid = lax.axis_index("c")
    sid = lax.axis_index("s")
    wid = sid * NC + cid
    base0 = wid * EPW

    def fill_ones(i, _):
        ones_v[pl.ds(i * 16, 16)] = jnp.ones((16,), jnp.float32)
        return 0

    lax.fori_loop(0, K // 16, fill_ones, 0)
    pltpu.sync_copy(zeros_hbm.at[pl.ds(sid * RPS, RPS)],
                    acc_sh.at[pl.ds(sid * RPS, RPS)])
    plsc.subcore_barrier()

    def fire(j, b):
        pltpu.async_copy(dst_hbm.at[pl.ds(base0 + j * K, K)],
                         dst_v.at[b], isem.at[b])

    def drain_and_scatter(j, b):
        pltpu.make_async_copy(dst_hbm.at[pl.ds(base0 + j * K, K)],
                              dst_v.at[b], isem.at[b]).wait()
        pltpu.sync_copy(ones_v, acc_sh.at[dst_v.at[b]], add=True)

    for b in range(CBUF):
        fire(b, b)

    def body(g, _):
        for b in range(CBUF):
            j = g * CBUF + b
            drain_and_scatter(j, b)
            fire(j + CBUF, b)
        return 0

    lax.fori_loop(0, CRING, body, 0)
    for b in range(CBUF):
        drain_and_scatter(CRING * CBUF + b, b)
    plsc.subcore_barrier()
    pltpu.sync_copy(acc_sh.at[pl.ds(sid * RPS, RPS)],
                    out_hbm.at[cid, pl.ds(sid * RPS, RPS)])


ABUF = 4


@functools.partial(
    pl.kernel,
    out_type=jax.ShapeDtypeStruct((NC, N_PAD, D), jnp.float32),
    mesh=_mesh,
    scratch_types=[
        pltpu.VMEM((ABUF, K), jnp.int32),
        pltpu.VMEM((ABUF, K), jnp.int32),
        pltpu.VMEM((ABUF, K, D), jnp.float32),
        pltpu.SemaphoreType.DMA((ABUF,)),
        pltpu.SemaphoreType.DMA((ABUF,)),
        pltpu.VMEM_SHARED((N_PAD, D), jnp.float32),
    ],
)
def _sc_agg(h_hbm, src_hbm, dst_hbm, zeros_hbm, out_hbm,
            src_v, dst_v, rows_v, isem, gsem, acc_sh):
    cid = lax.axis_index("c")
    sid = lax.axis_index("s")
    wid = sid * NC + cid
    base0 = wid * EPW
    pltpu.sync_copy(zeros_hbm.at[pl.ds(sid * RPS, RPS)],
                    acc_sh.at[pl.ds(sid * RPS, RPS)])
    plsc.subcore_barrier()

    def fire_idx(j, b):
        pltpu.async_copy(src_hbm.at[pl.ds(base0 + j * K, K)],
                         src_v.at[b], isem.at[b])
        pltpu.async_copy(dst_hbm.at[pl.ds(base0 + j * K, K)],
                         dst_v.at[b], isem.at[b])

    def wait_idx(j, b):
        pltpu.make_async_copy(src_hbm.at[pl.ds(base0 + j * K, K)],
                              src_v.at[b], isem.at[b]).wait()
        pltpu.make_async_copy(dst_hbm.at[pl.ds(base0 + j * K, K)],
                              dst_v.at[b], isem.at[b]).wait()

    def fire_gather(b):
        pltpu.async_copy(h_hbm.at[src_v.at[b]], rows_v.at[b], gsem.at[b])

    def drain_and_scatter(b):
        pltpu.make_async_copy(h_hbm.at[src_v.at[b]], rows_v.at[b],
                              gsem.at[b]).wait()
        pltpu.sync_copy(rows_v.at[b], acc_sh.at[dst_v.at[b]], add=True)

    def visit(j, b, next_gather, next_idx):
        if next_gather:
            b2 = (b + ABUF - 1) % ABUF
            wait_idx(j + ABUF - 1, b2)
            fire_gather(b2)
        drain_and_scatter(b)
        if next_idx:
            fire_idx(j + ABUF, b)

    for b in range(ABUF - 1):
        fire_idx(b, b)
    for b in range(ABUF - 1):
        wait_idx(b, b)
        fire_gather(b)
    fire_idx(ABUF - 1, ABUF - 1)

    def body(g, _):
        for b in range(ABUF):
            visit(g * ABUF + b, b, True, True)
        return 0

    lax.fori_loop(0, NB // ABUF - 1, body, 0)
    for b in range(ABUF):
        j = NB - ABUF + b
        visit(j, b, j + ABUF - 1 < NB, False)
    plsc.subcore_barrier()
    pltpu.sync_copy(acc_sh.at[pl.ds(sid * RPS, RPS)],
                    out_hbm.at[cid, pl.ds(sid * RPS, RPS)])


BN = 1000
GRID = N // BN

_row2 = lambda g: (g, 0)
_row3 = lambda g: (0, g, 0)
_full = lambda g: (0, 0)


def _t12_body(x_ref, w_ref, cnt_ref, hs_ref, dinv_ref):
    dinv = lax.rsqrt(1.0 + cnt_ref[...])
    hw = jnp.dot(x_ref[...], w_ref[...], preferred_element_type=jnp.float32)
    hs_ref[...] = hw * dinv
    dinv_ref[...] = dinv


def _tc_stage1(x, W1, cnt_col):
    return pl.pallas_call(
        _t12_body,
        grid=(GRID,),
        in_specs=[
            pl.BlockSpec((BN, D), _row2),
            pl.BlockSpec((D, D), _full),
            pl.BlockSpec((BN, 1), _row2),
        ],
        out_specs=[
            pl.BlockSpec((BN, D), _row2),
            pl.BlockSpec((BN, 1), _row2),
        ],
        out_shape=[
            jax.ShapeDtypeStruct((N, D), jnp.float32),
            jax.ShapeDtypeStruct((N, 1), jnp.float32),
        ],
    )(x, W1, cnt_col)


def _tmid_body(agg_ref, hs_ref, dinv_ref, b_ref, w_ref, out_ref):
    dinv = dinv_ref[...]
    h = dinv * (agg_ref[0] + agg_ref[1] + hs_ref[...]) + b_ref[...]
    h = jnp.maximum(h, 0.0)
    hw = jnp.dot(h, w_ref[...], preferred_element_type=jnp.float32)
    out_ref[...] = hw * dinv


def _tc_mid(agg, hs, dinv_col, b_row, W):
    return pl.pallas_call(
        _tmid_body,
        grid=(GRID,),
        in_specs=[
            pl.BlockSpec((NC, BN, D), _row3),
            pl.BlockSpec((BN, D), _row2),
            pl.BlockSpec((BN, 1), _row2),
            pl.BlockSpec((1, D), _full),
            pl.BlockSpec((D, D), _full),
        ],
        out_specs=pl.BlockSpec((BN, D), _row2),
        out_shape=jax.ShapeDtypeStruct((N, D), jnp.float32),
    )(agg, hs, dinv_col, b_row, W)


def _t4_body(agg_ref, hs_ref, dinv_ref, b_ref, out_ref):
    dinv = dinv_ref[...]
    h = dinv * (agg_ref[0] + agg_ref[1] + hs_ref[...]) + b_ref[...]
    out_ref[...] = dinv * jnp.maximum(h, 0.0)


def _tc_pre3(agg, hs2, dinv_col, b2_row):
    return pl.pallas_call(
        _t4_body,
        grid=(GRID,),
        in_specs=[
            pl.BlockSpec((NC, BN, D), _row3),
            pl.BlockSpec((BN, D), _row2),
            pl.BlockSpec((BN, 1), _row2),
            pl.BlockSpec((1, D), _full),
        ],
        out_specs=pl.BlockSpec((BN, D), _row2),
        out_shape=jax.ShapeDtypeStruct((N, D), jnp.float32),
    )(agg, hs2, dinv_col, b2_row)


def _t5_body(agg_ref, g_ref, dinv_ref, w_ref, b_ref, out_ref):
    z = dinv_ref[...] * (agg_ref[0] + agg_ref[1] + g_ref[...])
    out_ref[...] = (jnp.dot(z, w_ref[...], preferred_element_type=jnp.float32)
                    + b_ref[...])


def _tc_final(agg, g, dinv_col, W3p, b3_row):
    return pl.pallas_call(
        _t5_body,
        grid=(GRID,),
        in_specs=[
            pl.BlockSpec((NC, BN, D), _row3),
            pl.BlockSpec((BN, D), _row2),
            pl.BlockSpec((BN, 1), _row2),
            pl.BlockSpec((D, DP3), _full),
            pl.BlockSpec((1, DP3), _full),
        ],
        out_specs=pl.BlockSpec((BN, DP3), _row2),
        out_shape=jax.ShapeDtypeStruct((N, DP3), jnp.float32),
    )(agg, g, dinv_col, W3p, b3_row)


def kernel(x, edge_index, W1, b1, W2, b2, W3, b3):
    W3p = jnp.pad(W3, ((0, 0), (0, DP3 - W3.shape[1])))
    b1r = b1.reshape(1, D)
    b2r = b2.reshape(1, D)
    b3r = jnp.pad(b3, (0, DP3 - b3.shape[0])).reshape(1, DP3)
    z1 = jnp.zeros((N_PAD,), jnp.float32)
    z128 = jnp.zeros((N_PAD, D), jnp.float32)

    npad = E_PAD - E
    pi = jnp.arange(npad, dtype=jnp.int32)
    src = jnp.concatenate([edge_index[0], (pi * 131) % N])
    dst = jnp.concatenate([edge_index[1], N + pi % (N_PAD - N)])

    cnt_parts = _sc_count(dst, z1)
    cnt_col = (cnt_parts[0] + cnt_parts[1])[:N].reshape(N, 1)

    hs1, dinv_col = _tc_stage1(x, W1, cnt_col)

    agg1 = _sc_agg(hs1, src, dst, z128)
    hs2 = _tc_mid(agg1, hs1, dinv_col, b1r, W2)

    agg2 = _sc_agg(hs2, src, dst, z128)
    g = _tc_pre3(agg2, hs2, dinv_col, b2r)

    agg3 = _sc_agg(g, src, dst, z128)
    out16 = _tc_final(agg3, g, dinv_col, W3p, b3r)

    return out16[:, :2]


# speedup vs baseline: 1.2036x; 1.2036x over previous
import functools

import jax
import jax.numpy as jnp
from jax import lax
from jax.experimental import pallas as pl
from jax.experimental.pallas import tpu as pltpu
import jax.experimental.pallas.tpu_sc as plsc

N = 10000
E = 320000
D = 128
DP3 = 16
N_PAD = 10240
NC = 2
NS = 16
NW = NC * NS
EPW = 10240
E_PAD = NW * EPW
K = 80
NB = EPW // K
RPS = N_PAD // NS

_mesh = plsc.VectorSubcoreMesh(core_axis_name="c", subcore_axis_name="s")


CBUF = 4
CRING = NB // CBUF - 1


@functools.partial(
    pl.kernel,
    out_type=jax.ShapeDtypeStruct((NC, N_PAD), jnp.float32),
    mesh=_mesh,
    scratch_types=[
        pltpu.VMEM((CBUF, K), jnp.int32),
        pltpu.VMEM((K,), jnp.float32),
        pltpu.SemaphoreType.DMA((CBUF,)),
        pltpu.VMEM_SHARED((N_PAD,), jnp.float32),
    ],
)
def _sc_count(dst_hbm, zeros_hbm, out_hbm, dst_v, ones_v, isem, acc_sh):
    cid = lax.axis_index("c")
    sid = lax.axis_index("s")
    wid = sid * NC + cid
    base0 = wid * EPW

    def fill_ones(i, _):
        ones_v[pl.ds(i * 16, 16)] = jnp.ones((16,), jnp.float32)
        return 0

    lax.fori_loop(0, K // 16, fill_ones, 0)
    pltpu.sync_copy(zeros_hbm.at[pl.ds(sid * RPS, RPS)],
                    acc_sh.at[pl.ds(sid * RPS, RPS)])
    plsc.subcore_barrier()

    def fire(j, b):
        pltpu.async_copy(dst_hbm.at[pl.ds(base0 + j * K, K)],
                         dst_v.at[b], isem.at[b])

    def drain_and_scatter(j, b):
        pltpu.make_async_copy(dst_hbm.at[pl.ds(base0 + j * K, K)],
                              dst_v.at[b], isem.at[b]).wait()
        pltpu.sync_copy(ones_v, acc_sh.at[dst_v.at[b]], add=True)

    for b in range(CBUF):
        fire(b, b)

    def body(g, _):
        for b in range(CBUF):
            j = g * CBUF + b
            drain_and_scatter(j, b)
            fire(j + CBUF, b)
        return 0

    lax.fori_loop(0, CRING, body, 0)
    for b in range(CBUF):
        drain_and_scatter(CRING * CBUF + b, b)
    plsc.subcore_barrier()
    pltpu.sync_copy(acc_sh.at[pl.ds(sid * RPS, RPS)],
                    out_hbm.at[cid, pl.ds(sid * RPS, RPS)])


GK = 64
SK = 128
NS_B = EPW // SK


@functools.partial(
    pl.kernel,
    out_type=jax.ShapeDtypeStruct((NC, N_PAD, D), jnp.float32),
    mesh=_mesh,
    scratch_types=[
        pltpu.VMEM((EPW,), jnp.int32),
        pltpu.VMEM((2, SK), jnp.int32),
        pltpu.VMEM((2, SK, D), jnp.float32),
        pltpu.SemaphoreType.DMA((2,)),
        pltpu.SemaphoreType.DMA((2,)),
        pltpu.VMEM_SHARED((N_PAD, D), jnp.float32),
    ],
)
def _sc_agg(h_hbm, src_hbm, dst_hbm, zeros_hbm, out_hbm,
            src_all, dst_v, rows_v, isem, gsem, acc_sh):
    cid = lax.axis_index("c")
    sid = lax.axis_index("s")
    wid = sid * NC + cid
    base0 = wid * EPW
    pltpu.sync_copy(zeros_hbm.at[pl.ds(sid * RPS, RPS)],
                    acc_sh.at[pl.ds(sid * RPS, RPS)])
    pltpu.sync_copy(src_hbm.at[pl.ds(base0, EPW)], src_all)
    plsc.subcore_barrier()

    def fire_idx(s, b):
        pltpu.async_copy(dst_hbm.at[pl.ds(base0 + s * SK, SK)],
                         dst_v.at[b], isem.at[b])

    def wait_idx(s, b):
        pltpu.make_async_copy(dst_hbm.at[pl.ds(base0 + s * SK, SK)],
                              dst_v.at[b], isem.at[b]).wait()

    def fire_gathers(s, b):
        for h in range(2):
            pltpu.async_copy(
                h_hbm.at[src_all.at[pl.ds((2 * s + h) * GK, GK)]],
                rows_v.at[b, pl.ds(h * GK, GK)], gsem.at[b])

    def drain_and_scatter(s, b):
        for h in range(2):
            pltpu.make_async_copy(
                h_hbm.at[src_all.at[pl.ds((2 * s + h) * GK, GK)]],
                rows_v.at[b, pl.ds(h * GK, GK)], gsem.at[b]).wait()
        wait_idx(s, b)
        pltpu.sync_copy(rows_v.at[b], acc_sh.at[dst_v.at[b]], add=True)

    def visit(s, b, next_gather, next_idx):
        if next_gather:
            fire_gathers(s + 1, b ^ 1)
        drain_and_scatter(s, b)
        if next_idx:
            fire_idx(s + 2, b)

    fire_idx(0, 0)
    fire_idx(1, 1)
    fire_gathers(0, 0)

    def body(g, _):
        visit(2 * g, 0, True, True)
        visit(2 * g + 1, 1, True, True)
        return 0

    lax.fori_loop(0, NS_B // 2 - 1, body, 0)
    visit(NS_B - 2, 0, True, False)
    visit(NS_B - 1, 1, False, False)
    plsc.subcore_barrier()
    pltpu.sync_copy(acc_sh.at[pl.ds(sid * RPS, RPS)],
                    out_hbm.at[cid, pl.ds(sid * RPS, RPS)])


BN = 1000
GRID = N // BN

_row2 = lambda g: (g, 0)
_row3 = lambda g: (0, g, 0)
_full = lambda g: (0, 0)


def _t12_body(x_ref, w_ref, cnt_ref, hs_ref, dinv_ref):
    dinv = lax.rsqrt(1.0 + cnt_ref[...])
    hw = jnp.dot(x_ref[...], w_ref[...], preferred_element_type=jnp.float32)
    hs_ref[...] = hw * dinv
    dinv_ref[...] = dinv


def _tc_stage1(x, W1, cnt_col):
    return pl.pallas_call(
        _t12_body,
        grid=(GRID,),
        in_specs=[
            pl.BlockSpec((BN, D), _row2),
            pl.BlockSpec((D, D), _full),
            pl.BlockSpec((BN, 1), _row2),
        ],
        out_specs=[
            pl.BlockSpec((BN, D), _row2),
            pl.BlockSpec((BN, 1), _row2),
        ],
        out_shape=[
            jax.ShapeDtypeStruct((N, D), jnp.float32),
            jax.ShapeDtypeStruct((N, 1), jnp.float32),
        ],
    )(x, W1, cnt_col)


def _tmid_body(agg_ref, hs_ref, dinv_ref, b_ref, w_ref, out_ref):
    dinv = dinv_ref[...]
    h = dinv * (agg_ref[0] + agg_ref[1] + hs_ref[...]) + b_ref[...]
    h = jnp.maximum(h, 0.0)
    hw = jnp.dot(h, w_ref[...], preferred_element_type=jnp.float32)
    out_ref[...] = hw * dinv


def _tc_mid(agg, hs, dinv_col, b_row, W):
    return pl.pallas_call(
        _tmid_body,
        grid=(GRID,),
        in_specs=[
            pl.BlockSpec((NC, BN, D), _row3),
            pl.BlockSpec((BN, D), _row2),
            pl.BlockSpec((BN, 1), _row2),
            pl.BlockSpec((1, D), _full),
            pl.BlockSpec((D, D), _full),
        ],
        out_specs=pl.BlockSpec((BN, D), _row2),
        out_shape=jax.ShapeDtypeStruct((N, D), jnp.float32),
    )(agg, hs, dinv_col, b_row, W)


def _t4_body(agg_ref, hs_ref, dinv_ref, b_ref, out_ref):
    dinv = dinv_ref[...]
    h = dinv * (agg_ref[0] + agg_ref[1] + hs_ref[...]) + b_ref[...]
    out_ref[...] = dinv * jnp.maximum(h, 0.0)


def _tc_pre3(agg, hs2, dinv_col, b2_row):
    return pl.pallas_call(
        _t4_body,
        grid=(GRID,),
        in_specs=[
            pl.BlockSpec((NC, BN, D), _row3),
            pl.BlockSpec((BN, D), _row2),
            pl.BlockSpec((BN, 1), _row2),
            pl.BlockSpec((1, D), _full),
        ],
        out_specs=pl.BlockSpec((BN, D), _row2),
        out_shape=jax.ShapeDtypeStruct((N, D), jnp.float32),
    )(agg, hs2, dinv_col, b2_row)


def _t5_body(agg_ref, g_ref, dinv_ref, w_ref, b_ref, out_ref):
    z = dinv_ref[...] * (agg_ref[0] + agg_ref[1] + g_ref[...])
    out_ref[...] = (jnp.dot(z, w_ref[...], preferred_element_type=jnp.float32)
                    + b_ref[...])


def _tc_final(agg, g, dinv_col, W3p, b3_row):
    return pl.pallas_call(
        _t5_body,
        grid=(GRID,),
        in_specs=[
            pl.BlockSpec((NC, BN, D), _row3),
            pl.BlockSpec((BN, D), _row2),
            pl.BlockSpec((BN, 1), _row2),
            pl.BlockSpec((D, DP3), _full),
            pl.BlockSpec((1, DP3), _full),
        ],
        out_specs=pl.BlockSpec((BN, DP3), _row2),
        out_shape=jax.ShapeDtypeStruct((N, DP3), jnp.float32),
    )(agg, g, dinv_col, W3p, b3_row)


def kernel(x, edge_index, W1, b1, W2, b2, W3, b3):
    W3p = jnp.pad(W3, ((0, 0), (0, DP3 - W3.shape[1])))
    b1r = b1.reshape(1, D)
    b2r = b2.reshape(1, D)
    b3r = jnp.pad(b3, (0, DP3 - b3.shape[0])).reshape(1, DP3)
    z1 = jnp.zeros((N_PAD,), jnp.float32)
    z128 = jnp.zeros((N_PAD, D), jnp.float32)

    npad = E_PAD - E
    pi = jnp.arange(npad, dtype=jnp.int32)
    src = jnp.concatenate([edge_index[0], (pi * 131) % N])
    dst = jnp.concatenate([edge_index[1], N + pi % (N_PAD - N)])

    cnt_parts = _sc_count(dst, z1)
    cnt_col = (cnt_parts[0] + cnt_parts[1])[:N].reshape(N, 1)

    hs1, dinv_col = _tc_stage1(x, W1, cnt_col)

    agg1 = _sc_agg(hs1, src, dst, z128)
    hs2 = _tc_mid(agg1, hs1, dinv_col, b1r, W2)

    agg2 = _sc_agg(hs2, src, dst, z128)
    g = _tc_pre3(agg2, hs2, dinv_col, b2r)

    agg3 = _sc_agg(g, src, dst, z128)
    out16 = _tc_final(agg3, g, dinv_col, W3p, b3r)

    return out16[:, :2]
